# baseline (speedup 1.0000x reference)
import numpy as np
import jax
import jax.numpy as jnp
from jax.experimental import pallas as pl
from jax.experimental.pallas import tpu as pltpu

SEQ, BATCH = 128, 32
HID = 384
D_T, D_V, D_A, D_S = 1024, 342, 300, 768
WIN_P, WIN_F = 5, 5
SHIFT_WIN = 10
LRELU_SLOPE = 0.01

_TS = 8
_BB = 8


def _band_matrix() -> np.ndarray:
    t = np.arange(SEQ)
    deg = np.minimum(t, WIN_P) + np.minimum(SEQ - 1 - t, WIN_F) + 1.0
    band = (np.abs(t[:, None] - t[None, :]) <= WIN_P).astype(np.float64)
    C = band / np.sqrt(deg[:, None] * deg[None, :])
    return C.astype(np.float32)


_CMAT = _band_matrix()


def _lrelu(x):
    return jnp.where(x >= 0, x, LRELU_SLOPE * x)


def _enc_kernel(t0, t1, t2, t3, c4, c5, c0, c7, c8, v, a, q,
                Wte, bte, role, Wcin, bcin,
                Wzr, bzr, Wh, bh, Wgf, bg, Wv, bv, Wa, ba,
                fr_o, xzr_o, xh_o, gf_o, ev_o, ea_o):
    R = _TS * BATCH
    bf16 = jnp.bfloat16
    flat = lambda ref: ref[...].reshape(R, ref.shape[-1])
    flatb = lambda ref: flat(ref).astype(bf16)
    dot = lambda x, w: jnp.dot(x, w, preferred_element_type=jnp.float32)
    dotb = lambda x, w: jnp.dot(x, w.astype(bf16), preferred_element_type=jnp.float32)

    Wte_v = Wte[...]
    acc = (dotb(flatb(t0), Wte_v[0 * D_T:1 * D_T])
           + dotb(flatb(t1), Wte_v[1 * D_T:2 * D_T])
           + dotb(flatb(t2), Wte_v[2 * D_T:3 * D_T])
           + dotb(flatb(t3), Wte_v[3 * D_T:4 * D_T]))
    fr = _lrelu(acc + bte[...])
    q2 = q[...].reshape(R, 2)
    role_v = role[...]
    fr = fr + jnp.where(q2[:, 1:2] > q2[:, 0:1], role_v[1:2, :], role_v[0:1, :])

    Wcin_v = Wcin[...]
    ci = jnp.tanh(dotb(fr.astype(bf16), Wcin_v[0:HID])
                  + dotb(flatb(c4), Wcin_v[HID + 0 * D_S:HID + 1 * D_S])
                  + dotb(flatb(c5), Wcin_v[HID + 1 * D_S:HID + 2 * D_S])
                  + dotb(flatb(c0), Wcin_v[HID + 2 * D_S:HID + 3 * D_S])
                  + dotb(flatb(c7), Wcin_v[HID + 3 * D_S:HID + 4 * D_S])
                  + dotb(flatb(c8), Wcin_v[HID + 4 * D_S:HID + 5 * D_S])
                  + bcin[...])

    fr_o[...] = fr.reshape(_TS, BATCH, HID)
    doth = lambda x, w: jnp.dot(x, w, preferred_element_type=jnp.float32,
                                precision=jax.lax.Precision.HIGHEST)
    xzr_o[...] = (doth(ci, Wzr[...]) + bzr[...]).reshape(_TS, BATCH, 2 * HID)
    xh_o[...] = (doth(ci, Wh[...]) + bh[...]).reshape(_TS, BATCH, HID)
    gf_o[...] = (dotb(fr.astype(bf16), Wgf[...]) + bg[...]).reshape(_TS, BATCH, HID)
    ev_o[...] = _lrelu(dotb(flatb(v), Wv[...]) + bv[...]).reshape(_TS, BATCH, HID)
    ea_o[...] = _lrelu(dotb(flatb(a), Wa[...]) + ba[...]).reshape(_TS, BATCH, HID)


def _gru_kernel(xzr, xh, fr, gf, Uzr, Uh, Wgc, out, comet_scr):
    f32 = jnp.float32
    bf16 = jnp.bfloat16
    dot = lambda x, w: jnp.dot(x, w, preferred_element_type=f32)
    Uzr_v, Uh_v = Uzr[...], Uh[...]

    doth = lambda x, w: jnp.dot(x, w, preferred_element_type=f32,
                                precision=jax.lax.Precision.HIGHEST)

    def step(t, h):
        zr = jax.nn.sigmoid(xzr[t] + doth(h, Uzr_v))
        z, r = zr[:, :HID], zr[:, HID:]
        n = jnp.tanh(xh[t] + doth(r * h, Uh_v))
        hn = (1.0 - z) * n + z * h
        comet_scr[t] = hn
        return hn

    jax.lax.fori_loop(0, SEQ, step, jnp.zeros((BATCH, HID), f32))
    comet = comet_scr[...]
    pre = dot(comet.reshape(SEQ * BATCH, HID).astype(bf16), Wgc[...])
    g = jax.nn.sigmoid(gf[...] + pre.reshape(SEQ, BATCH, HID))
    out[...] = g * fr[...] + (1.0 - g) * comet


def _graph_kernel(et, ev, ea, Cm,
                  W00, b00, W01, b01, W10, b10, W11, b11, W20, b20, W21, b21,
                  Wfus, bfus, Wes, bes, W1tb, bsh1p, Wsh2, bsh2,
                  ffus_o, le_o, ls_o, sh_o):
    f32 = jnp.float32
    bf16 = jnp.bfloat16
    dot = lambda x, w: jnp.dot(x, w, preferred_element_type=f32)
    Cv = Cm[...]

    def bandmm(x):
        return dot(Cv, x.reshape(SEQ, _BB * HID)).reshape(SEQ, _BB, HID)

    def lin(x, W, b):
        return (dot(x.reshape(SEQ * _BB, HID), W[...]) + b[...]).reshape(SEQ, _BB, HID)

    def layer(xa, xb, W, b):
        na = xa + _lrelu(lin(bandmm(xb), W, b))
        nb = xb + _lrelu(lin(bandmm(xa), W, b))
        return na, nb

    xt, xv, xa_ = et[...], ev[...], ea[...]
    at, av, aa = bandmm(xt), bandmm(xv), bandmm(xa_)
    upd = lambda x, agg, W, b: x + _lrelu(lin(agg, W, b))
    tv_t, tv_v = upd(xt, av, W00, b00), upd(xv, at, W00, b00)
    ta_t, ta_a = upd(xt, aa, W10, b10), upd(xa_, at, W10, b10)
    va_v, va_a = upd(xv, aa, W20, b20), upd(xa_, av, W20, b20)
    tv_t, tv_v = layer(tv_t, tv_v, W01, b01)
    ta_t, ta_a = layer(ta_t, ta_a, W11, b11)
    va_v, va_a = layer(va_v, va_a, W21, b21)

    fsum = (_lrelu(lin(tv_t, Wfus, bfus)) + _lrelu(lin(ta_t, Wfus, bfus))
            + _lrelu(lin(tv_v, Wfus, bfus)) + _lrelu(lin(va_v, Wfus, bfus))
            + _lrelu(lin(ta_a, Wfus, bfus)) + _lrelu(lin(va_a, Wfus, bfus)))
    f = fsum * (1.0 / 6.0)
    ffus_o[...] = jnp.swapaxes(f, 0, 1).reshape(_BB * SEQ, HID)
    f2 = f.reshape(SEQ * _BB, HID)
    les = dot(f2, Wes[...]) + bes[...]
    ne = le_o.shape[-1]
    lesn = jnp.swapaxes(les.reshape(SEQ, _BB, -1), 0, 1).reshape(_BB * SEQ, -1)
    le_o[...] = lesn[:, :ne]
    ls_o[...] = lesn[:, ne:]

    AB = dot(f2, W1tb[...]) + bsh1p[...]
    A = AB[:, :HID].reshape(SEQ, _BB, HID)
    B = AB[:, HID:].reshape(SEQ, _BB, HID)
    Wsh2_v, bsh2_v = Wsh2[...], bsh2[...]
    for d in range(1, SHIFT_WIN + 1):
        Bs = jnp.concatenate([B[d:], jnp.zeros((d, _BB, HID), f32)], axis=0)
        Pv = _lrelu(A + Bs)
        l = dot(Pv.reshape(SEQ * _BB, HID), Wsh2_v) + bsh2_v
        sh_o[:, :, (d - 1) * 2:d * 2] = l.reshape(SEQ, _BB, 2)


def _full(shape):
    nd = len(shape)
    return pl.BlockSpec(shape, lambda *a: (0,) * nd)


def kernel(feature_t0, feature_t1, feature_t2, feature_t3,
           feature_c0, feature_c1, feature_c2, feature_c3, feature_c4,
           feature_c5, feature_c6, feature_c7, feature_c8,
           feature_v, feature_a, umask, qmask, dia_lengths, params):
    p = params
    r1 = lambda x: x.reshape(1, -1)
    f32 = jnp.float32

    blk = lambda shape: pl.BlockSpec((_TS,) + shape, lambda i: (i, 0, 0))
    seq_specs = [blk((BATCH, D_T))] * 4 + [blk((BATCH, D_S))] * 5 + [
        blk((BATCH, D_V)), blk((BATCH, D_A)), blk((BATCH, 2))]
    Wg = p['W_gate']
    bf = jnp.bfloat16
    Wzr = jnp.concatenate([p['Wz'], p['Wr']], axis=1)
    bzr = jnp.concatenate([p['bz'], p['br']]).reshape(1, -1)
    w_args = (p['W_te'], r1(p['b_te']), p['role_emb'], p['W_cin'], r1(p['b_cin']),
              Wzr, bzr, p['Wh'], r1(p['bh']), Wg[:HID], r1(p['b_gate']),
              p['W_v'], r1(p['b_v']), p['W_a'], r1(p['b_a']))
    out_bhh = jax.ShapeDtypeStruct((SEQ, BATCH, HID), f32)
    out_bh2 = jax.ShapeDtypeStruct((SEQ, BATCH, 2 * HID), f32)
    fr, xzr, xh, gf, ev, ea = pl.pallas_call(
        _enc_kernel,
        grid=(SEQ // _TS,),
        in_specs=seq_specs + [_full(tuple(w.shape)) for w in w_args],
        out_specs=[blk((BATCH, HID)), blk((BATCH, 2 * HID))] + [blk((BATCH, HID))] * 4,
        out_shape=[out_bhh, out_bh2, out_bhh, out_bhh, out_bhh, out_bhh],
        compiler_params=pltpu.CompilerParams(
            vmem_limit_bytes=64 * 1024 * 1024),
    )(feature_t0, feature_t1, feature_t2, feature_t3,
      feature_c4, feature_c5, feature_c0, feature_c7, feature_c8,
      feature_v, feature_a, qmask, *w_args)

    Uzr = jnp.concatenate([p['Uz'], p['Ur']], axis=1)
    gru_args = (xzr, xh, fr, gf, Uzr, p['Uh'], Wg[HID:].astype(bf))
    emo_t = pl.pallas_call(
        _gru_kernel,
        in_specs=[_full(tuple(w.shape)) for w in gru_args],
        out_specs=_full((SEQ, BATCH, HID)),
        out_shape=out_bhh,
        scratch_shapes=[pltpu.VMEM((SEQ, BATCH, HID), f32)],
    )(*gru_args)

    n_emo, n_sen = p['b_emo'].shape[0], p['b_sen'].shape[0]
    Wes = jnp.concatenate([p['W_emo'], p['W_sen']], axis=1)
    bes = jnp.concatenate([p['b_emo'], p['b_sen']]).reshape(1, -1)
    bsh1p = jnp.concatenate([p['b_sh1'], jnp.zeros((HID,), f32)]).reshape(1, -1)
    Wsh1 = p['W_sh1']
    W1tb = jnp.concatenate([Wsh1[:HID], Wsh1[HID:]], axis=1)
    g_w = (_CMAT,
           p['W_h0_0'], r1(p['b_h0_0']), p['W_h0_1'], r1(p['b_h0_1']),
           p['W_h1_0'], r1(p['b_h1_0']), p['W_h1_1'], r1(p['b_h1_1']),
           p['W_h2_0'], r1(p['b_h2_0']), p['W_h2_1'], r1(p['b_h2_1']),
           p['W_fus'], r1(p['b_fus']), Wes, bes,
           W1tb, bsh1p, p['W_sh2'], r1(p['b_sh2']))
    bblk = lambda w: pl.BlockSpec((SEQ, _BB, w), lambda j: (0, j, 0))
    ffus, le, ls, sh = pl.pallas_call(
        _graph_kernel,
        grid=(BATCH // _BB,),
        in_specs=[bblk(HID)] * 3 + [_full(tuple(w.shape)) for w in g_w],
        out_specs=[pl.BlockSpec((_BB * SEQ, HID), lambda j: (j, 0)),
                   pl.BlockSpec((_BB * SEQ, n_emo), lambda j: (j, 0)),
                   pl.BlockSpec((_BB * SEQ, n_sen), lambda j: (j, 0)),
                   bblk(2 * SHIFT_WIN)],
        out_shape=[jax.ShapeDtypeStruct((BATCH * SEQ, HID), f32),
                   jax.ShapeDtypeStruct((BATCH * SEQ, n_emo), f32),
                   jax.ShapeDtypeStruct((BATCH * SEQ, n_sen), f32),
                   jax.ShapeDtypeStruct((SEQ, BATCH, 2 * SHIFT_WIN), f32)],
    )(emo_t, ev, ea, *g_w)

    logit_emo, logit_sen, feat_fusion = le, ls, ffus
    S2 = jnp.transpose(sh, (1, 0, 2)).reshape(BATCH, SEQ * SHIFT_WIN, 2)
    parts = [S2[:, :1180]] + [S2[:, i * SHIFT_WIN:i * SHIFT_WIN + (SEQ - 1 - i)]
                              for i in range(118, 127)]
    logit_shift = jnp.concatenate(parts, axis=1).reshape(-1, 2)
    return logit_emo, logit_sen, logit_shift, feat_fusion


# speedup vs baseline: 1.4781x; 1.4781x over previous
import numpy as np
import jax
import jax.numpy as jnp
from jax.experimental import pallas as pl
from jax.experimental.pallas import tpu as pltpu

SEQ, BATCH = 128, 32
HID = 384
D_T, D_V, D_A, D_S = 1024, 342, 300, 768
WIN_P, WIN_F = 5, 5
SHIFT_WIN = 10
LRELU_SLOPE = 0.01

_TS = 8
_BB = 8


def _band_matrix() -> np.ndarray:
    t = np.arange(SEQ)
    deg = np.minimum(t, WIN_P) + np.minimum(SEQ - 1 - t, WIN_F) + 1.0
    band = (np.abs(t[:, None] - t[None, :]) <= WIN_P).astype(np.float64)
    C = band / np.sqrt(deg[:, None] * deg[None, :])
    return C.astype(np.float32)


_CMAT = _band_matrix()


def _lrelu(x):
    return jnp.where(x >= 0, x, LRELU_SLOPE * x)


def _enc_kernel(t0, t1, t2, t3, c4, c5, c0, c7, c8, v, a, q,
                Wte, bte, role, Wcin, bcin,
                Wzr, bzr, Wh, bh, Wgf, bg, Wv, bv, Wa, ba,
                fr_o, xzr_o, xh_o, gf_o, ev_o, ea_o):
    R = _TS * BATCH
    bf16 = jnp.bfloat16
    flat = lambda ref: ref[...].reshape(R, ref.shape[-1])
    flatb = lambda ref: flat(ref).astype(bf16)
    dot = lambda x, w: jnp.dot(x, w, preferred_element_type=jnp.float32)
    dotb = lambda x, w: jnp.dot(x, w.astype(bf16), preferred_element_type=jnp.float32)

    Wte_v = Wte[...]
    acc = (dotb(flatb(t0), Wte_v[0 * D_T:1 * D_T])
           + dotb(flatb(t1), Wte_v[1 * D_T:2 * D_T])
           + dotb(flatb(t2), Wte_v[2 * D_T:3 * D_T])
           + dotb(flatb(t3), Wte_v[3 * D_T:4 * D_T]))
    fr = _lrelu(acc + bte[...])
    q2 = q[...].reshape(R, 2)
    role_v = role[...]
    fr = fr + jnp.where(q2[:, 1:2] > q2[:, 0:1], role_v[1:2, :], role_v[0:1, :])

    Wcin_v = Wcin[...]
    ci = jnp.tanh(dotb(fr.astype(bf16), Wcin_v[0:HID])
                  + dotb(flatb(c4), Wcin_v[HID + 0 * D_S:HID + 1 * D_S])
                  + dotb(flatb(c5), Wcin_v[HID + 1 * D_S:HID + 2 * D_S])
                  + dotb(flatb(c0), Wcin_v[HID + 2 * D_S:HID + 3 * D_S])
                  + dotb(flatb(c7), Wcin_v[HID + 3 * D_S:HID + 4 * D_S])
                  + dotb(flatb(c8), Wcin_v[HID + 4 * D_S:HID + 5 * D_S])
                  + bcin[...])

    fr_o[...] = fr.reshape(_TS, BATCH, HID)
    cib = ci.astype(bf16)
    xzr_o[...] = (dotb(cib, Wzr[...]) + bzr[...]).reshape(_TS, BATCH, 2 * HID)
    xh_o[...] = (dotb(cib, Wh[...]) + bh[...]).reshape(_TS, BATCH, HID)
    gf_o[...] = (dotb(fr.astype(bf16), Wgf[...]) + bg[...]).reshape(_TS, BATCH, HID)
    ev_o[...] = _lrelu(dotb(flatb(v), Wv[...]) + bv[...]).reshape(_TS, BATCH, HID)
    ea_o[...] = _lrelu(dotb(flatb(a), Wa[...]) + ba[...]).reshape(_TS, BATCH, HID)


def _gru_kernel(xzr, xh, fr, gf, Uzr, Uh, Wgc, out, comet_scr):
    f32 = jnp.float32
    bf16 = jnp.bfloat16
    dot = lambda x, w: jnp.dot(x, w, preferred_element_type=f32)
    Uzr_v, Uh_v = Uzr[...], Uh[...]

    def step(t, h):
        zr = jax.nn.sigmoid(xzr[t] + dot(h.astype(bf16), Uzr_v))
        z, r = zr[:, :HID], zr[:, HID:]
        n = jnp.tanh(xh[t] + dot((r * h).astype(bf16), Uh_v))
        hn = (1.0 - z) * n + z * h
        comet_scr[t] = hn
        return hn

    jax.lax.fori_loop(0, SEQ, step, jnp.zeros((BATCH, HID), f32))
    comet = comet_scr[...]
    pre = dot(comet.reshape(SEQ * BATCH, HID).astype(bf16), Wgc[...])
    g = jax.nn.sigmoid(gf[...] + pre.reshape(SEQ, BATCH, HID))
    out[...] = g * fr[...] + (1.0 - g) * comet


def _graph_kernel(et, ev, ea, Cm,
                  W00, b00, W01, b01, W10, b10, W11, b11, W20, b20, W21, b21,
                  Wfus, bfus, Wes, bes, W1tb, bsh1p, Wsh2, bsh2,
                  ffus_o, le_o, ls_o, sh_o):
    f32 = jnp.float32
    bf16 = jnp.bfloat16
    dot = lambda x, w: jnp.dot(x, w, preferred_element_type=f32)
    Cv = Cm[...]

    def bandmm(x):
        return dot(Cv, x.reshape(SEQ, _BB * HID)).reshape(SEQ, _BB, HID)

    def lin(x, W, b):
        return (dot(x.reshape(SEQ * _BB, HID), W[...]) + b[...]).reshape(SEQ, _BB, HID)

    def layer(xa, xb, W, b):
        na = xa + _lrelu(lin(bandmm(xb), W, b))
        nb = xb + _lrelu(lin(bandmm(xa), W, b))
        return na, nb

    xt, xv, xa_ = et[...], ev[...], ea[...]
    at, av, aa = bandmm(xt), bandmm(xv), bandmm(xa_)
    upd = lambda x, agg, W, b: x + _lrelu(lin(agg, W, b))
    tv_t, tv_v = upd(xt, av, W00, b00), upd(xv, at, W00, b00)
    ta_t, ta_a = upd(xt, aa, W10, b10), upd(xa_, at, W10, b10)
    va_v, va_a = upd(xv, aa, W20, b20), upd(xa_, av, W20, b20)
    tv_t, tv_v = layer(tv_t, tv_v, W01, b01)
    ta_t, ta_a = layer(ta_t, ta_a, W11, b11)
    va_v, va_a = layer(va_v, va_a, W21, b21)

    fsum = (_lrelu(lin(tv_t, Wfus, bfus)) + _lrelu(lin(ta_t, Wfus, bfus))
            + _lrelu(lin(tv_v, Wfus, bfus)) + _lrelu(lin(va_v, Wfus, bfus))
            + _lrelu(lin(ta_a, Wfus, bfus)) + _lrelu(lin(va_a, Wfus, bfus)))
    f = fsum * (1.0 / 6.0)
    ffus_o[...] = jnp.swapaxes(f, 0, 1).reshape(_BB * SEQ, HID)
    f2 = f.reshape(SEQ * _BB, HID)
    les = dot(f2, Wes[...]) + bes[...]
    ne = le_o.shape[-1]
    lesn = jnp.swapaxes(les.reshape(SEQ, _BB, -1), 0, 1).reshape(_BB * SEQ, -1)
    le_o[...] = lesn[:, :ne]
    ls_o[...] = lesn[:, ne:]

    AB = dot(f2, W1tb[...]) + bsh1p[...]
    A = AB[:, :HID].reshape(SEQ, _BB, HID)
    B = AB[:, HID:].reshape(SEQ, _BB, HID)
    Wsh2_v, bsh2_v = Wsh2[...], bsh2[...]
    for d in range(1, SHIFT_WIN + 1):
        Bs = jnp.concatenate([B[d:], jnp.zeros((d, _BB, HID), f32)], axis=0)
        Pv = _lrelu(A + Bs)
        l = dot(Pv.reshape(SEQ * _BB, HID), Wsh2_v) + bsh2_v
        sh_o[:, :, (d - 1) * 2:d * 2] = l.reshape(SEQ, _BB, 2)


def _full(shape):
    nd = len(shape)
    return pl.BlockSpec(shape, lambda *a: (0,) * nd)


def kernel(feature_t0, feature_t1, feature_t2, feature_t3,
           feature_c0, feature_c1, feature_c2, feature_c3, feature_c4,
           feature_c5, feature_c6, feature_c7, feature_c8,
           feature_v, feature_a, umask, qmask, dia_lengths, params):
    p = params
    r1 = lambda x: x.reshape(1, -1)
    f32 = jnp.float32

    blk = lambda shape: pl.BlockSpec((_TS,) + shape, lambda i: (i, 0, 0))
    seq_specs = [blk((BATCH, D_T))] * 4 + [blk((BATCH, D_S))] * 5 + [
        blk((BATCH, D_V)), blk((BATCH, D_A)), blk((BATCH, 2))]
    Wg = p['W_gate']
    bf = jnp.bfloat16
    Wzr = jnp.concatenate([p['Wz'], p['Wr']], axis=1)
    bzr = jnp.concatenate([p['bz'], p['br']]).reshape(1, -1)
    w_args = (p['W_te'], r1(p['b_te']), p['role_emb'], p['W_cin'], r1(p['b_cin']),
              Wzr, bzr, p['Wh'], r1(p['bh']), Wg[:HID], r1(p['b_gate']),
              p['W_v'], r1(p['b_v']), p['W_a'], r1(p['b_a']))
    out_bhh = jax.ShapeDtypeStruct((SEQ, BATCH, HID), f32)
    out_bh2 = jax.ShapeDtypeStruct((SEQ, BATCH, 2 * HID), f32)
    fr, xzr, xh, gf, ev, ea = pl.pallas_call(
        _enc_kernel,
        grid=(SEQ // _TS,),
        in_specs=seq_specs + [_full(tuple(w.shape)) for w in w_args],
        out_specs=[blk((BATCH, HID)), blk((BATCH, 2 * HID))] + [blk((BATCH, HID))] * 4,
        out_shape=[out_bhh, out_bh2, out_bhh, out_bhh, out_bhh, out_bhh],
        compiler_params=pltpu.CompilerParams(
            vmem_limit_bytes=64 * 1024 * 1024),
    )(feature_t0, feature_t1, feature_t2, feature_t3,
      feature_c4, feature_c5, feature_c0, feature_c7, feature_c8,
      feature_v, feature_a, qmask, *w_args)

    Uzr = jnp.concatenate([p['Uz'], p['Ur']], axis=1).astype(bf)
    gru_args = (xzr, xh, fr, gf, Uzr, p['Uh'].astype(bf), Wg[HID:].astype(bf))
    emo_t = pl.pallas_call(
        _gru_kernel,
        in_specs=[_full(tuple(w.shape)) for w in gru_args],
        out_specs=_full((SEQ, BATCH, HID)),
        out_shape=out_bhh,
        scratch_shapes=[pltpu.VMEM((SEQ, BATCH, HID), f32)],
    )(*gru_args)

    n_emo, n_sen = p['b_emo'].shape[0], p['b_sen'].shape[0]
    Wes = jnp.concatenate([p['W_emo'], p['W_sen']], axis=1)
    bes = jnp.concatenate([p['b_emo'], p['b_sen']]).reshape(1, -1)
    bsh1p = jnp.concatenate([p['b_sh1'], jnp.zeros((HID,), f32)]).reshape(1, -1)
    Wsh1 = p['W_sh1']
    W1tb = jnp.concatenate([Wsh1[:HID], Wsh1[HID:]], axis=1)
    g_w = (_CMAT,
           p['W_h0_0'], r1(p['b_h0_0']), p['W_h0_1'], r1(p['b_h0_1']),
           p['W_h1_0'], r1(p['b_h1_0']), p['W_h1_1'], r1(p['b_h1_1']),
           p['W_h2_0'], r1(p['b_h2_0']), p['W_h2_1'], r1(p['b_h2_1']),
           p['W_fus'], r1(p['b_fus']), Wes, bes,
           W1tb, bsh1p, p['W_sh2'], r1(p['b_sh2']))
    bblk = lambda w: pl.BlockSpec((SEQ, _BB, w), lambda j: (0, j, 0))
    ffus, le, ls, sh = pl.pallas_call(
        _graph_kernel,
        grid=(BATCH // _BB,),
        in_specs=[bblk(HID)] * 3 + [_full(tuple(w.shape)) for w in g_w],
        out_specs=[pl.BlockSpec((_BB * SEQ, HID), lambda j: (j, 0)),
                   pl.BlockSpec((_BB * SEQ, n_emo), lambda j: (j, 0)),
                   pl.BlockSpec((_BB * SEQ, n_sen), lambda j: (j, 0)),
                   bblk(2 * SHIFT_WIN)],
        out_shape=[jax.ShapeDtypeStruct((BATCH * SEQ, HID), f32),
                   jax.ShapeDtypeStruct((BATCH * SEQ, n_emo), f32),
                   jax.ShapeDtypeStruct((BATCH * SEQ, n_sen), f32),
                   jax.ShapeDtypeStruct((SEQ, BATCH, 2 * SHIFT_WIN), f32)],
    )(emo_t, ev, ea, *g_w)

    logit_emo, logit_sen, feat_fusion = le, ls, ffus
    S2 = jnp.transpose(sh, (1, 0, 2)).reshape(BATCH, SEQ * SHIFT_WIN, 2)
    parts = [S2[:, :1180]] + [S2[:, i * SHIFT_WIN:i * SHIFT_WIN + (SEQ - 1 - i)]
                              for i in range(118, 127)]
    logit_shift = jnp.concatenate(parts, axis=1).reshape(-1, 2)
    return logit_emo, logit_sen, logit_shift, feat_fusion


# speedup vs baseline: 1.4865x; 1.0057x over previous
import numpy as np
import jax
import jax.numpy as jnp
from jax.experimental import pallas as pl
from jax.experimental.pallas import tpu as pltpu

SEQ, BATCH = 128, 32
HID = 384
D_T, D_V, D_A, D_S = 1024, 342, 300, 768
WIN_P, WIN_F = 5, 5
SHIFT_WIN = 10
LRELU_SLOPE = 0.01

_TS = 8
_BB = 8


def _band_matrix() -> np.ndarray:
    t = np.arange(SEQ)
    deg = np.minimum(t, WIN_P) + np.minimum(SEQ - 1 - t, WIN_F) + 1.0
    band = (np.abs(t[:, None] - t[None, :]) <= WIN_P).astype(np.float64)
    C = band / np.sqrt(deg[:, None] * deg[None, :])
    return C.astype(np.float32)


_CMAT = _band_matrix()


def _lrelu(x):
    return jnp.where(x >= 0, x, LRELU_SLOPE * x)


def _enc_kernel(t0, t1, t2, t3, c4, c5, c0, c7, c8, v, a, q,
                Wte, bte, role, Wcin, bcin,
                Wzr, bzr, Wh, bh, Wgf, bg, Wv, bv, Wa, ba,
                fr_o, xzr_o, xh_o, gf_o, ev_o, ea_o):
    R = _TS * BATCH
    bf16 = jnp.bfloat16
    flat = lambda ref: ref[...].reshape(R, ref.shape[-1])
    flatb = lambda ref: flat(ref).astype(bf16)
    dot = lambda x, w: jnp.dot(x, w, preferred_element_type=jnp.float32)
    dotb = lambda x, w: jnp.dot(x, w.astype(bf16), preferred_element_type=jnp.float32)

    Wte_v = Wte[...]
    acc = (dotb(flatb(t0), Wte_v[0 * D_T:1 * D_T])
           + dotb(flatb(t1), Wte_v[1 * D_T:2 * D_T])
           + dotb(flatb(t2), Wte_v[2 * D_T:3 * D_T])
           + dotb(flatb(t3), Wte_v[3 * D_T:4 * D_T]))
    fr = _lrelu(acc + bte[...])
    q2 = q[...].reshape(R, 2)
    role_v = role[...]
    fr = fr + jnp.where(q2[:, 1:2] > q2[:, 0:1], role_v[1:2, :], role_v[0:1, :])

    Wcin_v = Wcin[...]
    ci = jnp.tanh(dotb(fr.astype(bf16), Wcin_v[0:HID])
                  + dotb(flatb(c4), Wcin_v[HID + 0 * D_S:HID + 1 * D_S])
                  + dotb(flatb(c5), Wcin_v[HID + 1 * D_S:HID + 2 * D_S])
                  + dotb(flatb(c0), Wcin_v[HID + 2 * D_S:HID + 3 * D_S])
                  + dotb(flatb(c7), Wcin_v[HID + 3 * D_S:HID + 4 * D_S])
                  + dotb(flatb(c8), Wcin_v[HID + 4 * D_S:HID + 5 * D_S])
                  + bcin[...])

    fr_o[...] = fr.reshape(_TS, BATCH, HID)
    cib = ci.astype(bf16)
    xzr_o[...] = (dotb(cib, Wzr[...]) + bzr[...]).reshape(_TS, BATCH, 2 * HID)
    xh_o[...] = (dotb(cib, Wh[...]) + bh[...]).reshape(_TS, BATCH, HID)
    gf_o[...] = (dotb(fr.astype(bf16), Wgf[...]) + bg[...]).reshape(_TS, BATCH, HID)
    ev_o[...] = _lrelu(dotb(flatb(v), Wv[...]) + bv[...]).reshape(_TS, BATCH, HID)
    ea_o[...] = _lrelu(dotb(flatb(a), Wa[...]) + ba[...]).reshape(_TS, BATCH, HID)


def _gru_kernel(xzr, xh, fr, gf, Uzr, Uh, Wgc, out, comet_scr):
    f32 = jnp.float32
    bf16 = jnp.bfloat16
    dot = lambda x, w: jnp.dot(x, w, preferred_element_type=f32)
    Uzr_v, Uh_v = Uzr[...], Uh[...]

    def one(t, h):
        zr = jax.nn.sigmoid(xzr[t] + dot(h.astype(bf16), Uzr_v))
        z, r = zr[:, :HID], zr[:, HID:]
        n = jnp.tanh(xh[t] + dot((r * h).astype(bf16), Uh_v))
        hn = (1.0 - z) * n + z * h
        comet_scr[t] = hn
        return hn

    def step(i, h):
        return one(2 * i + 1, one(2 * i, h))

    jax.lax.fori_loop(0, SEQ // 2, step, jnp.zeros((BATCH, HID), f32))
    comet = comet_scr[...]
    pre = dot(comet.reshape(SEQ * BATCH, HID).astype(bf16), Wgc[...])
    g = jax.nn.sigmoid(gf[...] + pre.reshape(SEQ, BATCH, HID))
    out[...] = g * fr[...] + (1.0 - g) * comet


def _graph_kernel(et, ev, ea, Cm,
                  W00, b00, W01, b01, W10, b10, W11, b11, W20, b20, W21, b21,
                  Wfus, bfus, Wes, bes, W1tb, bsh1p, Wsh2, bsh2,
                  ffus_o, le_o, ls_o, sh_o):
    f32 = jnp.float32
    bf16 = jnp.bfloat16
    dot = lambda x, w: jnp.dot(x, w, preferred_element_type=f32)
    Cv = Cm[...]

    def bandmm(x):
        return dot(Cv, x.reshape(SEQ, _BB * HID)).reshape(SEQ, _BB, HID)

    def lin(x, W, b):
        return (dot(x.reshape(SEQ * _BB, HID), W[...]) + b[...]).reshape(SEQ, _BB, HID)

    def layer(xa, xb, W, b):
        na = xa + _lrelu(lin(bandmm(xb), W, b))
        nb = xb + _lrelu(lin(bandmm(xa), W, b))
        return na, nb

    xt, xv, xa_ = et[...], ev[...], ea[...]
    at, av, aa = bandmm(xt), bandmm(xv), bandmm(xa_)
    upd = lambda x, agg, W, b: x + _lrelu(lin(agg, W, b))
    tv_t, tv_v = upd(xt, av, W00, b00), upd(xv, at, W00, b00)
    ta_t, ta_a = upd(xt, aa, W10, b10), upd(xa_, at, W10, b10)
    va_v, va_a = upd(xv, aa, W20, b20), upd(xa_, av, W20, b20)
    tv_t, tv_v = layer(tv_t, tv_v, W01, b01)
    ta_t, ta_a = layer(ta_t, ta_a, W11, b11)
    va_v, va_a = layer(va_v, va_a, W21, b21)

    fsum = (_lrelu(lin(tv_t, Wfus, bfus)) + _lrelu(lin(ta_t, Wfus, bfus))
            + _lrelu(lin(tv_v, Wfus, bfus)) + _lrelu(lin(va_v, Wfus, bfus))
            + _lrelu(lin(ta_a, Wfus, bfus)) + _lrelu(lin(va_a, Wfus, bfus)))
    f = fsum * (1.0 / 6.0)
    ffus_o[...] = jnp.swapaxes(f, 0, 1).reshape(_BB * SEQ, HID)
    f2 = f.reshape(SEQ * _BB, HID)
    les = dot(f2, Wes[...]) + bes[...]
    ne = le_o.shape[-1]
    lesn = jnp.swapaxes(les.reshape(SEQ, _BB, -1), 0, 1).reshape(_BB * SEQ, -1)
    le_o[...] = lesn[:, :ne]
    ls_o[...] = lesn[:, ne:]

    AB = dot(f2, W1tb[...]) + bsh1p[...]
    A = AB[:, :HID].reshape(SEQ, _BB, HID)
    B = AB[:, HID:].reshape(SEQ, _BB, HID)
    Wsh2_v, bsh2_v = Wsh2[...], bsh2[...]
    for d in range(1, SHIFT_WIN + 1):
        Bs = jnp.concatenate([B[d:], jnp.zeros((d, _BB, HID), f32)], axis=0)
        Pv = _lrelu(A + Bs)
        l = dot(Pv.reshape(SEQ * _BB, HID), Wsh2_v) + bsh2_v
        sh_o[:, :, (d - 1) * 2:d * 2] = l.reshape(SEQ, _BB, 2)


def _full(shape):
    nd = len(shape)
    return pl.BlockSpec(shape, lambda *a: (0,) * nd)


def kernel(feature_t0, feature_t1, feature_t2, feature_t3,
           feature_c0, feature_c1, feature_c2, feature_c3, feature_c4,
           feature_c5, feature_c6, feature_c7, feature_c8,
           feature_v, feature_a, umask, qmask, dia_lengths, params):
    p = params
    r1 = lambda x: x.reshape(1, -1)
    f32 = jnp.float32

    blk = lambda shape: pl.BlockSpec((_TS,) + shape, lambda i: (i, 0, 0))
    seq_specs = [blk((BATCH, D_T))] * 4 + [blk((BATCH, D_S))] * 5 + [
        blk((BATCH, D_V)), blk((BATCH, D_A)), blk((BATCH, 2))]
    Wg = p['W_gate']
    bf = jnp.bfloat16
    Wzr = jnp.concatenate([p['Wz'], p['Wr']], axis=1)
    bzr = jnp.concatenate([p['bz'], p['br']]).reshape(1, -1)
    w_args = (p['W_te'], r1(p['b_te']), p['role_emb'], p['W_cin'], r1(p['b_cin']),
              Wzr, bzr, p['Wh'], r1(p['bh']), Wg[:HID], r1(p['b_gate']),
              p['W_v'], r1(p['b_v']), p['W_a'], r1(p['b_a']))
    out_bhh = jax.ShapeDtypeStruct((SEQ, BATCH, HID), f32)
    out_bh2 = jax.ShapeDtypeStruct((SEQ, BATCH, 2 * HID), f32)
    fr, xzr, xh, gf, ev, ea = pl.pallas_call(
        _enc_kernel,
        grid=(SEQ // _TS,),
        in_specs=seq_specs + [_full(tuple(w.shape)) for w in w_args],
        out_specs=[blk((BATCH, HID)), blk((BATCH, 2 * HID))] + [blk((BATCH, HID))] * 4,
        out_shape=[out_bhh, out_bh2, out_bhh, out_bhh, out_bhh, out_bhh],
        compiler_params=pltpu.CompilerParams(
            vmem_limit_bytes=64 * 1024 * 1024),
    )(feature_t0, feature_t1, feature_t2, feature_t3,
      feature_c4, feature_c5, feature_c0, feature_c7, feature_c8,
      feature_v, feature_a, qmask, *w_args)

    Uzr = jnp.concatenate([p['Uz'], p['Ur']], axis=1).astype(bf)
    gru_args = (xzr, xh, fr, gf, Uzr, p['Uh'].astype(bf), Wg[HID:].astype(bf))
    emo_t = pl.pallas_call(
        _gru_kernel,
        in_specs=[_full(tuple(w.shape)) for w in gru_args],
        out_specs=_full((SEQ, BATCH, HID)),
        out_shape=out_bhh,
        scratch_shapes=[pltpu.VMEM((SEQ, BATCH, HID), f32)],
    )(*gru_args)

    n_emo, n_sen = p['b_emo'].shape[0], p['b_sen'].shape[0]
    Wes = jnp.concatenate([p['W_emo'], p['W_sen']], axis=1)
    bes = jnp.concatenate([p['b_emo'], p['b_sen']]).reshape(1, -1)
    bsh1p = jnp.concatenate([p['b_sh1'], jnp.zeros((HID,), f32)]).reshape(1, -1)
    Wsh1 = p['W_sh1']
    W1tb = jnp.concatenate([Wsh1[:HID], Wsh1[HID:]], axis=1)
    g_w = (_CMAT,
           p['W_h0_0'], r1(p['b_h0_0']), p['W_h0_1'], r1(p['b_h0_1']),
           p['W_h1_0'], r1(p['b_h1_0']), p['W_h1_1'], r1(p['b_h1_1']),
           p['W_h2_0'], r1(p['b_h2_0']), p['W_h2_1'], r1(p['b_h2_1']),
           p['W_fus'], r1(p['b_fus']), Wes, bes,
           W1tb, bsh1p, p['W_sh2'], r1(p['b_sh2']))
    bblk = lambda w: pl.BlockSpec((SEQ, _BB, w), lambda j: (0, j, 0))
    ffus, le, ls, sh = pl.pallas_call(
        _graph_kernel,
        grid=(BATCH // _BB,),
        in_specs=[bblk(HID)] * 3 + [_full(tuple(w.shape)) for w in g_w],
        out_specs=[pl.BlockSpec((_BB * SEQ, HID), lambda j: (j, 0)),
                   pl.BlockSpec((_BB * SEQ, n_emo), lambda j: (j, 0)),
                   pl.BlockSpec((_BB * SEQ, n_sen), lambda j: (j, 0)),
                   bblk(2 * SHIFT_WIN)],
        out_shape=[jax.ShapeDtypeStruct((BATCH * SEQ, HID), f32),
                   jax.ShapeDtypeStruct((BATCH * SEQ, n_emo), f32),
                   jax.ShapeDtypeStruct((BATCH * SEQ, n_sen), f32),
                   jax.ShapeDtypeStruct((SEQ, BATCH, 2 * SHIFT_WIN), f32)],
    )(emo_t, ev, ea, *g_w)

    logit_emo, logit_sen, feat_fusion = le, ls, ffus
    S2 = jnp.transpose(sh, (1, 0, 2)).reshape(BATCH, SEQ * SHIFT_WIN, 2)
    parts = [S2[:, :1180]] + [S2[:, i * SHIFT_WIN:i * SHIFT_WIN + (SEQ - 1 - i)]
                              for i in range(118, 127)]
    logit_shift = jnp.concatenate(parts, axis=1).reshape(-1, 2)
    return logit_emo, logit_sen, logit_shift, feat_fusion
